# baseline (speedup 1.0000x reference)
import functools

import jax
import jax.numpy as jnp
from jax import lax
from jax.experimental import pallas as pl
from jax.experimental.pallas import tpu as pltpu
from jax.experimental.pallas import tpu_sc as plsc

_NT = 32


def _make_edge_kernel(n_edges, n_nodes, nch):
  ept = n_edges // _NT
  blk = 80
  chunk = 2000
  nchunks = ept // chunk
  rpc = chunk // blk
  ngroups = blk // 16
  nacc = nch + 1
  mesh = plsc.VectorSubcoreMesh(core_axis_name="c", subcore_axis_name="s")

  @functools.partial(
      pl.kernel,
      mesh=mesh,
      compiler_params=pltpu.CompilerParams(
          needs_layout_passes=False, use_tc_tiling_on_sc=False),
      out_type=jax.ShapeDtypeStruct((_NT, nacc, n_nodes), jnp.float32),
      scratch_types=[
          pltpu.VMEM((3, n_nodes), jnp.float32),
          pltpu.VMEM((nacc, n_nodes), jnp.float32),
          pltpu.VMEM((blk, 16), jnp.float32),
          pltpu.VMEM((blk, 16), jnp.float32),
          pltpu.VMEM((chunk,), jnp.int32),
          pltpu.VMEM((chunk,), jnp.int32),
          pltpu.VMEM((chunk,), jnp.float32),
          pltpu.SemaphoreType.DMA,
          pltpu.SemaphoreType.DMA,
      ],
  )
  def edge_kernel(src_hbm, dst_hbm, u_hbm, t_hbm, scal_hbm, p_hbm,
                  scal_v, acc_v, rows_a, rows_b, srcc_v, dstc_v, uc_v,
                  sem_a, sem_b):
    cid = lax.axis_index("c")
    sid = lax.axis_index("s")
    wid = sid * 2 + cid

    pltpu.sync_copy(scal_hbm, scal_v)

    zero16 = jnp.zeros((16,), jnp.float32)

    def zbody(i, carry):
      for c in range(nacc):
        acc_v[c, pl.ds(i * 16, 16)] = zero16
      return carry

    lax.fori_loop(0, n_nodes // 16, zbody, 0)

    iota16 = lax.iota(jnp.int32, 16)
    base0 = wid * ept
    c0 = jnp.zeros((16,), jnp.int32)

    def gcopy(r, rows_v, sem):
      return pltpu.make_async_copy(
          t_hbm.at[srcc_v.at[pl.ds(r * blk, blk)]], rows_v, sem)

    def compute(r, rows_v):
      for g in range(ngroups):
        off = r * blk + g * 16
        src16 = srcc_v[pl.ds(off, 16)]
        dst16 = dstc_v[pl.ds(off, 16)]
        b1 = uc_v[pl.ds(off, 16)]
        b0 = 1.0 - b1
        adst = plsc.load_gather(scal_v, [c0, dst16])
        s0 = plsc.load_gather(scal_v, [c0 + 1, src16])
        s1 = plsc.load_gather(scal_v, [c0 + 2, src16])
        logit = adst + b0 * s0 + b1 * s1
        logit = jnp.where(logit >= 0.0, logit, 0.2 * logit)
        ex = jnp.exp(jnp.minimum(logit, 60.0))
        eid = iota16 + g * 16
        plsc.addupdate_scatter(acc_v, [c0 + nch, dst16], ex)
        for h in range(nch):
          y0h = plsc.load_gather(rows_v, [eid, c0 + h])
          y1h = plsc.load_gather(rows_v, [eid, c0 + (8 + h)])
          mh = b0 * y0h + b1 * y1h
          plsc.addupdate_scatter(acc_v, [c0 + h, dst16], ex * mh)

    def cbody(c, carry):
      cbase = base0 + c * chunk
      pltpu.sync_copy(src_hbm.at[pl.ds(cbase, chunk)], srcc_v)
      pltpu.sync_copy(dst_hbm.at[pl.ds(cbase, chunk)], dstc_v)
      pltpu.sync_copy(u_hbm.at[pl.ds(cbase, chunk)], uc_v)

      gcopy(0, rows_a, sem_a).start()
      gcopy(1, rows_b, sem_b).start()

      def pbody(i, carry2):
        r0 = 2 * i
        gcopy(r0, rows_a, sem_a).wait()
        compute(r0, rows_a)

        @pl.when(r0 + 2 < rpc)
        def _():
          gcopy(r0 + 2, rows_a, sem_a).start()

        r1 = r0 + 1
        gcopy(r1, rows_b, sem_b).wait()
        compute(r1, rows_b)

        @pl.when(r1 + 2 < rpc)
        def _():
          gcopy(r1 + 2, rows_b, sem_b).start()

        return carry2

      lax.fori_loop(0, rpc // 2, pbody, 0)
      gcopy(rpc - 1, rows_a, sem_a).wait()
      compute(rpc - 1, rows_a)
      return carry

    lax.fori_loop(0, nchunks, cbody, 0)
    pltpu.sync_copy(acc_v, p_hbm.at[wid])

  return edge_kernel


def _proj1_tc(x, root1, wslim1, wscal1, n_nodes, d, bn=1024):
  grid = (pl.cdiv(n_nodes, bn),)

  def body(x_ref, root_ref, w_ref, ws_ref, t_ref, scal_ref, hp_ref):
    xb = x_ref[...]
    t_ref[...] = jnp.dot(xb, w_ref[...], preferred_element_type=jnp.float32)
    scal_ref[...] = lax.dot_general(
        ws_ref[...], xb, (((0,), (1,)), ((), ())),
        preferred_element_type=jnp.float32)
    hp_ref[...] = lax.dot_general(
        root_ref[...], xb, (((0,), (1,)), ((), ())),
        preferred_element_type=jnp.float32)

  return pl.pallas_call(
      body,
      grid=grid,
      in_specs=[
          pl.BlockSpec((bn, d), lambda i: (i, 0)),
          pl.BlockSpec((d, 8), lambda i: (0, 0)),
          pl.BlockSpec((d, 16), lambda i: (0, 0)),
          pl.BlockSpec((d, 3), lambda i: (0, 0)),
      ],
      out_specs=[
          pl.BlockSpec((bn, 16), lambda i: (i, 0)),
          pl.BlockSpec((3, bn), lambda i: (0, i)),
          pl.BlockSpec((8, bn), lambda i: (0, i)),
      ],
      out_shape=[
          jax.ShapeDtypeStruct((n_nodes, 16), jnp.float32),
          jax.ShapeDtypeStruct((3, n_nodes), jnp.float32),
          jax.ShapeDtypeStruct((8, n_nodes), jnp.float32),
      ],
  )(x, root1, wslim1, wscal1)


def _mid_tc(p1, hroot1_p, bias1, root2, wslim2, wscal2, n_nodes, bn=1024):
  grid = (pl.cdiv(n_nodes, bn),)

  def body(p_ref, hp_ref, b_ref, root_ref, w_ref, ws_ref,
           t_ref, scal_ref, hp2_ref):
    ps = jnp.sum(p_ref[...], axis=0)
    num = ps[0:8]
    den = ps[8:9]
    hp = num / (den + 1e-16) + hp_ref[...] + b_ref[...].reshape(8, 1)
    hp = jnp.where(hp > 0.0, hp, jnp.exp(hp) - 1.0)
    t_ref[...] = lax.dot_general(hp, w_ref[...], (((0,), (0,)), ((), ())),
                                 preferred_element_type=jnp.float32)
    scal_ref[...] = lax.dot_general(ws_ref[...], hp, (((0,), (0,)), ((), ())),
                                    preferred_element_type=jnp.float32)
    hp2_ref[...] = lax.dot_general(root_ref[...], hp, (((0,), (0,)), ((), ())),
                                   preferred_element_type=jnp.float32)

  return pl.pallas_call(
      body,
      grid=grid,
      in_specs=[
          pl.BlockSpec((_NT, 9, bn), lambda i: (0, 0, i)),
          pl.BlockSpec((8, bn), lambda i: (0, i)),
          pl.BlockSpec((8,), lambda i: (0,)),
          pl.BlockSpec((8, 7), lambda i: (0, 0)),
          pl.BlockSpec((8, 16), lambda i: (0, 0)),
          pl.BlockSpec((8, 3), lambda i: (0, 0)),
      ],
      out_specs=[
          pl.BlockSpec((bn, 16), lambda i: (i, 0)),
          pl.BlockSpec((3, bn), lambda i: (0, i)),
          pl.BlockSpec((7, bn), lambda i: (0, i)),
      ],
      out_shape=[
          jax.ShapeDtypeStruct((n_nodes, 16), jnp.float32),
          jax.ShapeDtypeStruct((3, n_nodes), jnp.float32),
          jax.ShapeDtypeStruct((7, n_nodes), jnp.float32),
      ],
  )(p1, hroot1_p, bias1, root2, wslim2, wscal2)


def _final_tc(p2, hroot2_p, bias2, n_nodes, bn=1024):
  grid = (pl.cdiv(n_nodes, bn),)

  def body(p_ref, hp_ref, b_ref, out_ref):
    ps = jnp.sum(p_ref[...], axis=0)
    num = ps[0:7]
    den = ps[7:8]
    z = num / (den + 1e-16) + hp_ref[...] + b_ref[...].reshape(7, 1)
    m = jnp.max(z, axis=0, keepdims=True)
    t = z - m
    lse = jnp.log(jnp.sum(jnp.exp(t), axis=0, keepdims=True))
    out_ref[...] = jnp.transpose(t - lse, (1, 0))

  return pl.pallas_call(
      body,
      grid=grid,
      in_specs=[
          pl.BlockSpec((_NT, 8, bn), lambda i: (0, 0, i)),
          pl.BlockSpec((7, bn), lambda i: (0, i)),
          pl.BlockSpec((7,), lambda i: (0,)),
      ],
      out_specs=pl.BlockSpec((bn, 7), lambda i: (i, 0)),
      out_shape=jax.ShapeDtypeStruct((n_nodes, 7), jnp.float32),
  )(p2, hroot2_p, bias2)


def kernel(x, edge_index, pseudo, W1, root1, bias1, att1, W2, root2, bias2,
           att2):
  n_nodes, d = x.shape
  n_edges = edge_index.shape[1]
  src = edge_index[0]
  dst = edge_index[1]
  u = pseudo[:, 0]

  attd1, attm1 = att1[:8], att1[8:]
  wslim1 = jnp.concatenate([W1[0], W1[1]], axis=1)
  wscal1 = jnp.stack(
      [root1 @ attd1, W1[0] @ attm1, W1[1] @ attm1], axis=1)
  attd2, attm2 = att2[:7], att2[7:]
  z8 = jnp.zeros((8, 1), jnp.float32)
  wslim2 = jnp.concatenate([W2[0], z8, W2[1], z8], axis=1)
  wscal2 = jnp.stack(
      [root2 @ attd2, W2[0] @ attm2, W2[1] @ attm2], axis=1)

  t1, scal1, hroot1_p = _proj1_tc(x, root1, wslim1, wscal1, n_nodes, d)
  p1 = _make_edge_kernel(n_edges, n_nodes, 8)(src, dst, u, t1, scal1)
  t2, scal2, hroot2_p = _mid_tc(
      p1, hroot1_p, bias1, root2, wslim2, wscal2, n_nodes)
  p2 = _make_edge_kernel(n_edges, n_nodes, 7)(src, dst, u, t2, scal2)
  return _final_tc(p2, hroot2_p, bias2, n_nodes)


# speedup vs baseline: 1.0367x; 1.0367x over previous
import functools

import jax
import jax.numpy as jnp
from jax import lax
from jax.experimental import pallas as pl
from jax.experimental.pallas import tpu as pltpu
from jax.experimental.pallas import tpu_sc as plsc

_NT = 32


def _make_edge_kernel(n_edges, n_nodes, nch):
  ept = n_edges // _NT
  blk = 80
  chunk = 2000
  nchunks = ept // chunk
  rpc = chunk // blk
  ngroups = blk // 16
  nacc = nch + 1
  mesh = plsc.VectorSubcoreMesh(core_axis_name="c", subcore_axis_name="s")

  @functools.partial(
      pl.kernel,
      mesh=mesh,
      compiler_params=pltpu.CompilerParams(
          needs_layout_passes=False, use_tc_tiling_on_sc=False),
      out_type=jax.ShapeDtypeStruct((_NT, nacc, n_nodes), jnp.float32),
      scratch_types=[
          pltpu.VMEM((3, n_nodes), jnp.float32),
          pltpu.VMEM((nacc, n_nodes), jnp.float32),
          pltpu.VMEM((blk, 16), jnp.float32),
          pltpu.VMEM((blk, 16), jnp.float32),
          pltpu.VMEM((blk, 16), jnp.float32),
          pltpu.VMEM((chunk,), jnp.int32),
          pltpu.VMEM((chunk,), jnp.int32),
          pltpu.VMEM((chunk,), jnp.float32),
          pltpu.SemaphoreType.DMA,
          pltpu.SemaphoreType.DMA,
          pltpu.SemaphoreType.DMA,
      ],
  )
  def edge_kernel(src_hbm, dst_hbm, u_hbm, t_hbm, scal_hbm, p_hbm,
                  scal_v, acc_v, rows_a, rows_b, rows_c, srcc_v, dstc_v, uc_v,
                  sem_a, sem_b, sem_c):
    cid = lax.axis_index("c")
    sid = lax.axis_index("s")
    wid = sid * 2 + cid

    pltpu.sync_copy(scal_hbm, scal_v)

    zero16 = jnp.zeros((16,), jnp.float32)

    def zbody(i, carry):
      for c in range(nacc):
        acc_v[c, pl.ds(i * 16, 16)] = zero16
      return carry

    lax.fori_loop(0, n_nodes // 16, zbody, 0)

    iota16 = lax.iota(jnp.int32, 16)
    base0 = wid * ept
    c0 = jnp.zeros((16,), jnp.int32)

    def gcopy(r, rows_v, sem):
      return pltpu.make_async_copy(
          t_hbm.at[srcc_v.at[pl.ds(r * blk, blk)]], rows_v, sem)

    def compute(r, rows_v):
      for g in range(ngroups):
        off = r * blk + g * 16
        src16 = srcc_v[pl.ds(off, 16)]
        dst16 = dstc_v[pl.ds(off, 16)]
        b1 = uc_v[pl.ds(off, 16)]
        b0 = 1.0 - b1
        adst = plsc.load_gather(scal_v, [c0, dst16])
        s0 = plsc.load_gather(scal_v, [c0 + 1, src16])
        s1 = plsc.load_gather(scal_v, [c0 + 2, src16])
        logit = adst + b0 * s0 + b1 * s1
        logit = jnp.where(logit >= 0.0, logit, 0.2 * logit)
        ex = jnp.exp(jnp.minimum(logit, 60.0))
        eid = iota16 + g * 16
        plsc.addupdate_scatter(acc_v, [c0 + nch, dst16], ex)
        for h in range(nch):
          y0h = plsc.load_gather(rows_v, [eid, c0 + h])
          y1h = plsc.load_gather(rows_v, [eid, c0 + (8 + h)])
          mh = b0 * y0h + b1 * y1h
          plsc.addupdate_scatter(acc_v, [c0 + h, dst16], ex * mh)

    def cbody(c, carry):
      cbase = base0 + c * chunk
      pltpu.sync_copy(src_hbm.at[pl.ds(cbase, chunk)], srcc_v)
      pltpu.sync_copy(dst_hbm.at[pl.ds(cbase, chunk)], dstc_v)
      pltpu.sync_copy(u_hbm.at[pl.ds(cbase, chunk)], uc_v)

      gcopy(0, rows_a, sem_a).start()
      gcopy(1, rows_b, sem_b).start()
      gcopy(2, rows_c, sem_c).start()

      def pbody(i, carry2):
        r0 = 3 * i
        for k, (rv, sm) in enumerate(
            [(rows_a, sem_a), (rows_b, sem_b), (rows_c, sem_c)]):
          r = r0 + k
          gcopy(r, rv, sm).wait()
          compute(r, rv)

          @pl.when(r + 3 < rpc)
          def _():
            gcopy(r + 3, rv, sm).start()

        return carry2

      lax.fori_loop(0, rpc // 3, pbody, 0)
      gcopy(rpc - 1, rows_a, sem_a).wait()
      compute(rpc - 1, rows_a)
      return carry

    lax.fori_loop(0, nchunks, cbody, 0)
    pltpu.sync_copy(acc_v, p_hbm.at[wid])

  return edge_kernel


def _proj1_tc(x, root1, wslim1, wscal1, n_nodes, d, bn=1024):
  grid = (pl.cdiv(n_nodes, bn),)

  def body(x_ref, root_ref, w_ref, ws_ref, t_ref, scal_ref, hp_ref):
    xb = x_ref[...]
    t_ref[...] = jnp.dot(xb, w_ref[...], preferred_element_type=jnp.float32)
    scal_ref[...] = lax.dot_general(
        ws_ref[...], xb, (((0,), (1,)), ((), ())),
        preferred_element_type=jnp.float32)
    hp_ref[...] = lax.dot_general(
        root_ref[...], xb, (((0,), (1,)), ((), ())),
        preferred_element_type=jnp.float32)

  return pl.pallas_call(
      body,
      grid=grid,
      in_specs=[
          pl.BlockSpec((bn, d), lambda i: (i, 0)),
          pl.BlockSpec((d, 8), lambda i: (0, 0)),
          pl.BlockSpec((d, 16), lambda i: (0, 0)),
          pl.BlockSpec((d, 3), lambda i: (0, 0)),
      ],
      out_specs=[
          pl.BlockSpec((bn, 16), lambda i: (i, 0)),
          pl.BlockSpec((3, bn), lambda i: (0, i)),
          pl.BlockSpec((8, bn), lambda i: (0, i)),
      ],
      out_shape=[
          jax.ShapeDtypeStruct((n_nodes, 16), jnp.float32),
          jax.ShapeDtypeStruct((3, n_nodes), jnp.float32),
          jax.ShapeDtypeStruct((8, n_nodes), jnp.float32),
      ],
  )(x, root1, wslim1, wscal1)


def _mid_tc(p1, hroot1_p, bias1, root2, wslim2, wscal2, n_nodes, bn=1024):
  grid = (pl.cdiv(n_nodes, bn),)

  def body(p_ref, hp_ref, b_ref, root_ref, w_ref, ws_ref,
           t_ref, scal_ref, hp2_ref):
    ps = jnp.sum(p_ref[...], axis=0)
    num = ps[0:8]
    den = ps[8:9]
    hp = num / (den + 1e-16) + hp_ref[...] + b_ref[...].reshape(8, 1)
    hp = jnp.where(hp > 0.0, hp, jnp.exp(hp) - 1.0)
    t_ref[...] = lax.dot_general(hp, w_ref[...], (((0,), (0,)), ((), ())),
                                 preferred_element_type=jnp.float32)
    scal_ref[...] = lax.dot_general(ws_ref[...], hp, (((0,), (0,)), ((), ())),
                                    preferred_element_type=jnp.float32)
    hp2_ref[...] = lax.dot_general(root_ref[...], hp, (((0,), (0,)), ((), ())),
                                   preferred_element_type=jnp.float32)

  return pl.pallas_call(
      body,
      grid=grid,
      in_specs=[
          pl.BlockSpec((_NT, 9, bn), lambda i: (0, 0, i)),
          pl.BlockSpec((8, bn), lambda i: (0, i)),
          pl.BlockSpec((8,), lambda i: (0,)),
          pl.BlockSpec((8, 7), lambda i: (0, 0)),
          pl.BlockSpec((8, 16), lambda i: (0, 0)),
          pl.BlockSpec((8, 3), lambda i: (0, 0)),
      ],
      out_specs=[
          pl.BlockSpec((bn, 16), lambda i: (i, 0)),
          pl.BlockSpec((3, bn), lambda i: (0, i)),
          pl.BlockSpec((7, bn), lambda i: (0, i)),
      ],
      out_shape=[
          jax.ShapeDtypeStruct((n_nodes, 16), jnp.float32),
          jax.ShapeDtypeStruct((3, n_nodes), jnp.float32),
          jax.ShapeDtypeStruct((7, n_nodes), jnp.float32),
      ],
  )(p1, hroot1_p, bias1, root2, wslim2, wscal2)


def _final_tc(p2, hroot2_p, bias2, n_nodes, bn=1024):
  grid = (pl.cdiv(n_nodes, bn),)

  def body(p_ref, hp_ref, b_ref, out_ref):
    ps = jnp.sum(p_ref[...], axis=0)
    num = ps[0:7]
    den = ps[7:8]
    z = num / (den + 1e-16) + hp_ref[...] + b_ref[...].reshape(7, 1)
    m = jnp.max(z, axis=0, keepdims=True)
    t = z - m
    lse = jnp.log(jnp.sum(jnp.exp(t), axis=0, keepdims=True))
    out_ref[...] = t - lse

  return pl.pallas_call(
      body,
      grid=grid,
      in_specs=[
          pl.BlockSpec((_NT, 8, bn), lambda i: (0, 0, i)),
          pl.BlockSpec((7, bn), lambda i: (0, i)),
          pl.BlockSpec((7,), lambda i: (0,)),
      ],
      out_specs=pl.BlockSpec((7, bn), lambda i: (0, i)),
      out_shape=jax.ShapeDtypeStruct((7, n_nodes), jnp.float32),
  )(p2, hroot2_p, bias2)


def kernel(x, edge_index, pseudo, W1, root1, bias1, att1, W2, root2, bias2,
           att2):
  n_nodes, d = x.shape
  n_edges = edge_index.shape[1]
  src = edge_index[0]
  dst = edge_index[1]
  u = pseudo[:, 0]

  attd1, attm1 = att1[:8], att1[8:]
  wslim1 = jnp.concatenate([W1[0], W1[1]], axis=1)
  wscal1 = jnp.stack(
      [root1 @ attd1, W1[0] @ attm1, W1[1] @ attm1], axis=1)
  attd2, attm2 = att2[:7], att2[7:]
  z8 = jnp.zeros((8, 1), jnp.float32)
  wslim2 = jnp.concatenate([W2[0], z8, W2[1], z8], axis=1)
  wscal2 = jnp.stack(
      [root2 @ attd2, W2[0] @ attm2, W2[1] @ attm2], axis=1)

  t1, scal1, hroot1_p = _proj1_tc(x, root1, wslim1, wscal1, n_nodes, d)
  p1 = _make_edge_kernel(n_edges, n_nodes, 8)(src, dst, u, t1, scal1)
  t2, scal2, hroot2_p = _mid_tc(
      p1, hroot1_p, bias1, root2, wslim2, wscal2, n_nodes)
  p2 = _make_edge_kernel(n_edges, n_nodes, 7)(src, dst, u, t2, scal2)
  return _final_tc(p2, hroot2_p, bias2, n_nodes).T
